# baseline (speedup 1.0000x reference)
import functools

import jax
import jax.numpy as jnp
from jax import lax
from jax.experimental import pallas as pl
from jax.experimental.pallas import tpu as pltpu
from jax.experimental.pallas import tpu_sc as plsc

V = 30523
NQ = 900
BQ = 64
GRID1 = 15
NROWS = BQ * GRID1
NPAD = 1024
K = 100
KPAD = 112
NOUT = 128
DUMP = 127
NJV = 57

NC = 2
NS = 16
NTILES = NC * NS
IPT = NPAD // NTILES


def _stage1_body(scale_ref, logits_ref, boxes_ref,
                 scores_ref, payload_ref):
    x = logits_ref[...]
    xnl = x[:, :V - 1]
    m_nl = jnp.max(xnl, axis=-1, keepdims=True)
    amax = jnp.argmax(xnl, axis=-1)[:, None]
    l_last = x[:, V - 1:V]
    m_all = jnp.maximum(m_nl, l_last)
    z = jnp.sum(jnp.exp(x - m_all), axis=-1, keepdims=True)
    row = pl.program_id(0) * BQ + lax.broadcasted_iota(jnp.int32, (BQ, 1), 0)
    valid = row < NQ
    scores_ref[...] = jnp.where(valid, jnp.exp(m_nl - m_all) / z, -1.0)
    b = boxes_ref[...]
    cx, cy, w, h = b[:, 0:1], b[:, 1:2], b[:, 2:3], b[:, 3:4]
    xyxy = jnp.concatenate(
        [cx - 0.5 * w, cy - 0.5 * h, cx + 0.5 * w, cy + 0.5 * h], axis=-1)
    out_score = 1.0 - jnp.exp(l_last - m_all) / z
    pad = jnp.zeros((BQ, 122), jnp.float32)
    payload = jnp.concatenate(
        [out_score, xyxy * scale_ref[...], amax.astype(jnp.float32), pad],
        axis=-1)
    payload_ref[...] = jnp.where(valid, payload, 0.0)


def _stage1(logits2d, boxes2d, scale):
    return pl.pallas_call(
        _stage1_body,
        grid=(GRID1,),
        in_specs=[
            pl.BlockSpec((1, 4), lambda i: (0, 0)),
            pl.BlockSpec((BQ, V), lambda i: (i, 0)),
            pl.BlockSpec((BQ, 4), lambda i: (i, 0)),
        ],
        out_specs=[
            pl.BlockSpec((BQ, 1), lambda i: (i, 0)),
            pl.BlockSpec((BQ, 128), lambda i: (i, 0)),
        ],
        out_shape=[
            jax.ShapeDtypeStruct((NROWS, 1), jnp.float32),
            jax.ShapeDtypeStruct((NROWS, 128), jnp.float32),
        ],
    )(scale, logits2d, boxes2d)


def _stage2_body(scores_hbm, payload_hbm, proj_hbm,
                 misc_hbm, proj_out_hbm,
                 sc_v, idx_v, tgt_v, pay_v, pr_v, sem):
    wid = lax.axis_index("s") * NC + lax.axis_index("c")
    i_base = wid * IPT

    pltpu.sync_copy(scores_hbm, sc_v)

    iota16 = lax.broadcasted_iota(jnp.int32, (16,), 0)
    one = jnp.full((16,), 1, jnp.int32)
    zero = jnp.zeros((16,), jnp.int32)

    for k in range(IPT // 16):
        lo = i_base + 16 * k
        vi = sc_v[pl.ds(lo, 16)]

        def count_body(strict, vi=vi):
            def body(t, cnt):
                c = cnt
                vj = sc_v[pl.ds(16 * t, 16)]
                for u in range(16):
                    sj = jnp.full((16,), vj[u])
                    hit = (sj > vi) if strict else (sj >= vi)
                    c = c + jnp.where(hit, one, zero)
                return c
            return body

        t_mid = wid * (IPT // 16) + k
        cnt = lax.fori_loop(0, t_mid, count_body(False),
                            jnp.zeros((16,), jnp.int32))
        vj = sc_v[pl.ds(lo, 16)]
        for u in range(16):
            sj = jnp.full((16,), vj[u])
            hit = (sj > vi) | ((sj == vi) & (iota16 > u))
            cnt = cnt + jnp.where(hit, one, zero)
        cnt = lax.fori_loop(t_mid + 1, NJV, count_body(True), cnt)

        ivec = iota16 + lo
        sel = cnt < KPAD
        tgt = jnp.where(sel, cnt, DUMP)
        gidx = jnp.where(sel, ivec, 0)
        idx_v[pl.ds(16 * k, 16)] = gidx
        tgt_v[pl.ds(16 * k, 16)] = tgt

    pltpu.async_copy(payload_hbm.at[idx_v], pay_v, sem).wait()
    pltpu.async_copy(pay_v, misc_hbm.at[tgt_v], sem).wait()
    pltpu.async_copy(proj_hbm.at[idx_v], pr_v, sem).wait()
    pltpu.async_copy(pr_v, proj_out_hbm.at[tgt_v], sem).wait()


@functools.partial(
    pl.kernel,
    out_type=[
        jax.ShapeDtypeStruct((NOUT, 128), jnp.float32),
        jax.ShapeDtypeStruct((NOUT, 256), jnp.float32),
    ],
    mesh=plsc.VectorSubcoreMesh(core_axis_name="c", subcore_axis_name="s"),
    scratch_types=[
        pltpu.VMEM((NPAD,), jnp.float32),
        pltpu.VMEM((IPT,), jnp.int32),
        pltpu.VMEM((IPT,), jnp.int32),
        pltpu.VMEM((IPT, 128), jnp.float32),
        pltpu.VMEM((IPT, 256), jnp.float32),
        pltpu.SemaphoreType.DMA,
    ],
)
def _stage2(scores_hbm, payload_hbm, proj_hbm,
            misc_hbm, proj_out_hbm, *scratch):
    _stage2_body(scores_hbm, payload_hbm, proj_hbm,
                 misc_hbm, proj_out_hbm, *scratch)


def kernel(pred_logits, pred_boxes, proj_queries, target_sizes):
    logits2d = pred_logits[0]
    boxes2d = pred_boxes[0]
    img_h = target_sizes[:, 0].astype(jnp.float32)
    img_w = target_sizes[:, 1].astype(jnp.float32)
    scale = jnp.stack([img_w, img_h, img_w, img_h], axis=1)

    scores_p, payload_p = _stage1(logits2d, boxes2d, scale)

    pad = NPAD - NROWS
    scores_f = jnp.pad(scores_p.reshape(NROWS), (0, pad), constant_values=-1.0)
    payload_f = jnp.pad(payload_p, ((0, pad), (0, 0)))

    misc, proj_sel = _stage2(scores_f, payload_f, proj_queries[0])

    scores = misc[None, :K, 0]
    labels = jnp.zeros((1, K), jnp.float32)
    boxes = misc[None, :K, 1:5]
    word_labels = misc[:K, 5].astype(jnp.int32)[None]
    proj_q = proj_sel[None, :K]
    return (scores, labels, boxes, word_labels, proj_q)


# speedup vs baseline: 1.0091x; 1.0091x over previous
import functools

import jax
import jax.numpy as jnp
from jax import lax
from jax.experimental import pallas as pl
from jax.experimental.pallas import tpu as pltpu
from jax.experimental.pallas import tpu_sc as plsc

V = 30523
NQ = 900
BQ = 64
GRID1 = 15
NROWS = BQ * GRID1
NPAD = 1024
K = 100
KPAD = 112
NOUT = 128
DUMP = 127
NJV = 57

NC = 2
NS = 16
NTILES = NC * NS
IPT = NPAD // NTILES


def _stage1_body(scale_ref, logits_ref, boxes_ref,
                 scores_ref, payload_ref):
    x = logits_ref[...]
    xnl = x[:, :V - 1]
    m_nl = jnp.max(xnl, axis=-1, keepdims=True)
    amax = jnp.argmax(xnl, axis=-1)[:, None]
    l_last = x[:, V - 1:V]
    m_all = jnp.maximum(m_nl, l_last)
    z = jnp.sum(jnp.exp(x - m_all), axis=-1, keepdims=True)
    row = pl.program_id(0) * BQ + lax.broadcasted_iota(jnp.int32, (BQ, 1), 0)
    valid = row < NQ
    scores_ref[...] = jnp.where(valid, jnp.exp(m_nl - m_all) / z, -1.0)
    b = boxes_ref[...]
    cx, cy, w, h = b[:, 0:1], b[:, 1:2], b[:, 2:3], b[:, 3:4]
    xyxy = jnp.concatenate(
        [cx - 0.5 * w, cy - 0.5 * h, cx + 0.5 * w, cy + 0.5 * h], axis=-1)
    out_score = 1.0 - jnp.exp(l_last - m_all) / z
    pad = jnp.zeros((BQ, 122), jnp.float32)
    payload = jnp.concatenate(
        [out_score, xyxy * scale_ref[...], amax.astype(jnp.float32), pad],
        axis=-1)
    payload_ref[...] = jnp.where(valid, payload, 0.0)


def _stage1(logits3d, boxes3d, scale):
    return pl.pallas_call(
        _stage1_body,
        grid=(GRID1,),
        in_specs=[
            pl.BlockSpec((1, 4), lambda i: (0, 0)),
            pl.BlockSpec((BQ, V), lambda i: (i, 0)),
            pl.BlockSpec((BQ, 4), lambda i: (i, 0)),
        ],
        out_specs=[
            pl.BlockSpec((BQ, 1), lambda i: (i, 0)),
            pl.BlockSpec((BQ, 128), lambda i: (i, 0)),
        ],
        out_shape=[
            jax.ShapeDtypeStruct((NPAD, 1), jnp.float32),
            jax.ShapeDtypeStruct((NPAD, 128), jnp.float32),
        ],
    )(scale, logits3d, boxes3d)


def _stage2_body(scores_hbm, payload_hbm, proj_hbm,
                 misc_hbm, proj_out_hbm,
                 sc_v, idx_v, tgt_v, pay_v, pr_v, sem):
    wid = lax.axis_index("s") * NC + lax.axis_index("c")
    i_base = wid * IPT

    pltpu.sync_copy(scores_hbm, sc_v)

    iota16 = lax.broadcasted_iota(jnp.int32, (16,), 0)
    one = jnp.full((16,), 1, jnp.int32)
    zero = jnp.zeros((16,), jnp.int32)

    for k in range(IPT // 16):
        lo = i_base + 16 * k
        vi = sc_v[pl.ds(lo, 16)]

        def count_body(strict, vi=vi):
            def body(t, cnt):
                c = cnt
                vj = sc_v[pl.ds(16 * t, 16)]
                for u in range(16):
                    sj = jnp.full((16,), vj[u])
                    hit = (sj > vi) if strict else (sj >= vi)
                    c = c + jnp.where(hit, one, zero)
                return c
            return body

        t_mid = wid * (IPT // 16) + k
        cnt = lax.fori_loop(0, t_mid, count_body(False),
                            jnp.zeros((16,), jnp.int32))
        vj = sc_v[pl.ds(lo, 16)]
        for u in range(16):
            sj = jnp.full((16,), vj[u])
            hit = (sj > vi) | ((sj == vi) & (iota16 > u))
            cnt = cnt + jnp.where(hit, one, zero)
        cnt = lax.fori_loop(t_mid + 1, NJV, count_body(True), cnt)

        ivec = iota16 + lo
        sel = (cnt < KPAD) & (ivec < NQ)
        tgt = jnp.where(sel, cnt, DUMP)
        gidx = jnp.where(sel, ivec, 0)
        idx_v[pl.ds(16 * k, 16)] = gidx
        tgt_v[pl.ds(16 * k, 16)] = tgt

    pltpu.async_copy(payload_hbm.at[idx_v], pay_v, sem).wait()
    pltpu.async_copy(pay_v, misc_hbm.at[tgt_v], sem).wait()
    pltpu.async_copy(proj_hbm.at[idx_v], pr_v, sem).wait()
    pltpu.async_copy(pr_v, proj_out_hbm.at[tgt_v], sem).wait()


@functools.partial(
    pl.kernel,
    out_type=[
        jax.ShapeDtypeStruct((NOUT, 128), jnp.float32),
        jax.ShapeDtypeStruct((NOUT, 256), jnp.float32),
    ],
    mesh=plsc.VectorSubcoreMesh(core_axis_name="c", subcore_axis_name="s"),
    scratch_types=[
        pltpu.VMEM((NPAD,), jnp.float32),
        pltpu.VMEM((IPT,), jnp.int32),
        pltpu.VMEM((IPT,), jnp.int32),
        pltpu.VMEM((IPT, 128), jnp.float32),
        pltpu.VMEM((IPT, 256), jnp.float32),
        pltpu.SemaphoreType.DMA,
    ],
)
def _stage2(scores_hbm, payload_hbm, proj_hbm,
            misc_hbm, proj_out_hbm, *scratch):
    _stage2_body(scores_hbm, payload_hbm, proj_hbm,
                 misc_hbm, proj_out_hbm, *scratch)


def kernel(pred_logits, pred_boxes, proj_queries, target_sizes):
    img_h = target_sizes[:, 0].astype(jnp.float32)
    img_w = target_sizes[:, 1].astype(jnp.float32)
    scale = jnp.stack([img_w, img_h, img_w, img_h], axis=1)

    scores_p, payload_p = _stage1(pred_logits[0], pred_boxes[0], scale)

    misc, proj_sel = _stage2(scores_p.reshape(NPAD), payload_p,
                             proj_queries[0])

    scores = misc[None, :K, 0]
    labels = jnp.zeros((1, K), jnp.float32)
    boxes = misc[None, :K, 1:5]
    word_labels = misc[:K, 5].astype(jnp.int32)[None]
    proj_q = proj_sel[None, :K]
    return (scores, labels, boxes, word_labels, proj_q)


# speedup vs baseline: 2.3760x; 2.3546x over previous
import functools

import jax
import jax.numpy as jnp
from jax import lax
from jax.experimental import pallas as pl
from jax.experimental.pallas import tpu as pltpu
from jax.experimental.pallas import tpu_sc as plsc

V = 30523
NQ = 900
BQ = 64
GRID1 = 15
NROWS = BQ * GRID1
NPAD = 1024
K = 100
KPAD = 112
NOUT = 128
DUMP = 127
NJV = 57

NC = 2
NS = 16
NTILES = NC * NS
IPT = NPAD // NTILES


def _stage1_body(scale_ref, logits_ref, boxes_ref,
                 scores_ref, payload_ref):
    x = logits_ref[...]
    xnl = x[:, :V - 1]
    m_nl = jnp.max(xnl, axis=-1, keepdims=True)
    amax = jnp.argmax(xnl, axis=-1)[:, None]
    l_last = x[:, V - 1:V]
    m_all = jnp.maximum(m_nl, l_last)
    z = jnp.sum(jnp.exp(x - m_all), axis=-1, keepdims=True)
    row = pl.program_id(0) * BQ + lax.broadcasted_iota(jnp.int32, (BQ, 1), 0)
    valid = row < NQ
    scores_ref[...] = jnp.where(valid, jnp.exp(m_nl - m_all) / z, -1.0)
    b = boxes_ref[...]
    cx, cy, w, h = b[:, 0:1], b[:, 1:2], b[:, 2:3], b[:, 3:4]
    xyxy = jnp.concatenate(
        [cx - 0.5 * w, cy - 0.5 * h, cx + 0.5 * w, cy + 0.5 * h], axis=-1)
    out_score = 1.0 - jnp.exp(l_last - m_all) / z
    pad = jnp.zeros((BQ, 122), jnp.float32)
    payload = jnp.concatenate(
        [out_score, xyxy * scale_ref[...], amax.astype(jnp.float32), pad],
        axis=-1)
    payload_ref[...] = jnp.where(valid, payload, 0.0)


def _stage1(logits3d, boxes3d, scale):
    return pl.pallas_call(
        _stage1_body,
        grid=(GRID1,),
        in_specs=[
            pl.BlockSpec((1, 4), lambda i: (0, 0)),
            pl.BlockSpec((None, BQ, V), lambda i: (0, i, 0)),
            pl.BlockSpec((None, BQ, 4), lambda i: (0, i, 0)),
        ],
        out_specs=[
            pl.BlockSpec((BQ, 1), lambda i: (i, 0)),
            pl.BlockSpec((BQ, 128), lambda i: (i, 0)),
        ],
        out_shape=[
            jax.ShapeDtypeStruct((NPAD, 1), jnp.float32),
            jax.ShapeDtypeStruct((NPAD, 128), jnp.float32),
        ],
    )(scale, logits3d, boxes3d)


def _stage2_body(scores_hbm, payload_hbm, proj_hbm,
                 misc_hbm, proj_out_hbm,
                 sc_v, idx_v, tgt_v, pay_v, pr_v, sem):
    wid = lax.axis_index("s") * NC + lax.axis_index("c")
    i_base = wid * IPT

    pltpu.sync_copy(scores_hbm, sc_v)

    iota16 = lax.broadcasted_iota(jnp.int32, (16,), 0)
    one = jnp.full((16,), 1, jnp.int32)
    zero = jnp.zeros((16,), jnp.int32)

    for k in range(IPT // 16):
        lo = i_base + 16 * k
        vi = sc_v[pl.ds(lo, 16)]

        def count_body(strict, vi=vi):
            def body(t, cnt):
                c = cnt
                vj = sc_v[pl.ds(16 * t, 16)]
                for u in range(16):
                    sj = jnp.full((16,), vj[u])
                    hit = (sj > vi) if strict else (sj >= vi)
                    c = c + jnp.where(hit, one, zero)
                return c
            return body

        t_mid = wid * (IPT // 16) + k
        cnt = lax.fori_loop(0, t_mid, count_body(False),
                            jnp.zeros((16,), jnp.int32))
        vj = sc_v[pl.ds(lo, 16)]
        for u in range(16):
            sj = jnp.full((16,), vj[u])
            hit = (sj > vi) | ((sj == vi) & (iota16 > u))
            cnt = cnt + jnp.where(hit, one, zero)
        cnt = lax.fori_loop(t_mid + 1, NJV, count_body(True), cnt)

        ivec = iota16 + lo
        sel = (cnt < KPAD) & (ivec < NQ)
        tgt = jnp.where(sel, cnt, DUMP)
        gidx = jnp.where(sel, ivec, 0)
        idx_v[pl.ds(16 * k, 16)] = gidx
        tgt_v[pl.ds(16 * k, 16)] = tgt

    pltpu.async_copy(payload_hbm.at[idx_v], pay_v, sem).wait()
    pltpu.async_copy(pay_v, misc_hbm.at[tgt_v], sem).wait()
    pltpu.async_copy(proj_hbm.at[idx_v], pr_v, sem).wait()
    pltpu.async_copy(pr_v, proj_out_hbm.at[tgt_v], sem).wait()


@functools.partial(
    pl.kernel,
    out_type=[
        jax.ShapeDtypeStruct((NOUT, 128), jnp.float32),
        jax.ShapeDtypeStruct((NOUT, 256), jnp.float32),
    ],
    mesh=plsc.VectorSubcoreMesh(core_axis_name="c", subcore_axis_name="s"),
    scratch_types=[
        pltpu.VMEM((NPAD,), jnp.float32),
        pltpu.VMEM((IPT,), jnp.int32),
        pltpu.VMEM((IPT,), jnp.int32),
        pltpu.VMEM((IPT, 128), jnp.float32),
        pltpu.VMEM((IPT, 256), jnp.float32),
        pltpu.SemaphoreType.DMA,
    ],
)
def _stage2(scores_hbm, payload_hbm, proj_hbm,
            misc_hbm, proj_out_hbm, *scratch):
    _stage2_body(scores_hbm, payload_hbm, proj_hbm,
                 misc_hbm, proj_out_hbm, *scratch)


def kernel(pred_logits, pred_boxes, proj_queries, target_sizes):
    img_h = target_sizes[:, 0].astype(jnp.float32)
    img_w = target_sizes[:, 1].astype(jnp.float32)
    scale = jnp.stack([img_w, img_h, img_w, img_h], axis=1)

    scores_p, payload_p = _stage1(pred_logits, pred_boxes, scale)

    misc, proj_sel = _stage2(scores_p.reshape(NPAD), payload_p,
                             proj_queries[0])

    scores = misc[None, :K, 0]
    labels = jnp.zeros((1, K), jnp.float32)
    boxes = misc[None, :K, 1:5]
    word_labels = misc[:K, 5].astype(jnp.int32)[None]
    proj_q = proj_sel[None, :K]
    return (scores, labels, boxes, word_labels, proj_q)
